# Initial kernel scaffold; baseline (speedup 1.0000x reference)
import jax
import jax.numpy as jnp
from jax.experimental import pallas as pl


def kernel(x, W_enc, b_enc, codebook, W_dec, b_dec):
    raise NotImplementedError("write your pallas kernel here")


# speedup vs baseline: 1.2664x; 1.2664x over previous
import functools

import jax
import jax.numpy as jnp
from jax import lax
from jax.experimental import pallas as pl
from jax.experimental.pallas import tpu as pltpu
from jax.experimental.pallas import tpu_sc as plsc


def _tc_body(x_ref, we_ref, be_ref, cb_ref, wd_ref, bd_ref,
             q_ref, loss_ref, table_ref):
    i = pl.program_id(0)
    cb = cb_ref[...]
    z = jnp.dot(x_ref[...], we_ref[...],
                preferred_element_type=jnp.float32) + be_ref[...]
    zsq = jnp.sum(z * z, axis=1, keepdims=True)
    csq = jnp.sum(cb * cb, axis=1)
    scores = lax.dot_general(z, cb, (((1,), (1,)), ((), ())),
                             preferred_element_type=jnp.float32)
    d2 = zsq - 2.0 * scores + csq[None, :]
    q_ref[0, 0, :] = jnp.argmin(d2, axis=1).astype(jnp.int32)
    part = jnp.sum(jnp.min(d2, axis=1)).reshape(1, 1)
    prev = jnp.where(i == 0, jnp.zeros((1, 1), jnp.float32), loss_ref[...])
    loss_ref[...] = prev + part

    @pl.when(i == 0)
    def _():
        table_ref[...] = jnp.dot(cb, wd_ref[...],
                                 preferred_element_type=jnp.float32) + bd_ref[...]


def _make_sc_gather(n_tokens, d_in, chunk):
    info = plsc.get_sparse_core_info()
    nc, ns = info.num_cores, info.num_subcores
    nw = nc * ns
    per_w = n_tokens // nw
    assert per_w % chunk == 0

    @functools.partial(
        pl.kernel,
        mesh=plsc.VectorSubcoreMesh(core_axis_name="c", subcore_axis_name="s"),
        out_type=jax.ShapeDtypeStruct((n_tokens, d_in), jnp.float32),
        scratch_types=[
            pltpu.VMEM((chunk,), jnp.int32),
            pltpu.VMEM((chunk, d_in), jnp.float32),
            pltpu.SemaphoreType.DMA,
        ],
        compiler_params=pltpu.CompilerParams(use_tc_tiling_on_sc=False),
    )
    def sc_gather(table_hbm, idx_hbm, out_hbm, idx_v, rows_v, sem):
        wid = lax.axis_index("s") * nc + lax.axis_index("c")
        base = wid * per_w
        for c in range(per_w // chunk):
            off = base + c * chunk
            pltpu.sync_copy(idx_hbm.at[pl.ds(off, chunk)], idx_v)
            pltpu.async_copy(table_hbm.at[idx_v], rows_v, sem).wait()
            pltpu.sync_copy(rows_v, out_hbm.at[pl.ds(off, chunk)])

    return sc_gather


def kernel(x, W_enc, b_enc, codebook, W_dec, b_dec):
    B, T, D_in = x.shape
    K, D_lat = codebook.shape
    N = B * T
    BLK = 2048
    nblk = N // BLK

    xf = x.reshape(N, D_in)
    q3, loss_sum, table = pl.pallas_call(
        _tc_body,
        grid=(nblk,),
        in_specs=[
            pl.BlockSpec((BLK, D_in), lambda i: (i, 0)),
            pl.BlockSpec((D_in, D_lat), lambda i: (0, 0)),
            pl.BlockSpec((1, D_lat), lambda i: (0, 0)),
            pl.BlockSpec((K, D_lat), lambda i: (0, 0)),
            pl.BlockSpec((D_lat, D_in), lambda i: (0, 0)),
            pl.BlockSpec((1, D_in), lambda i: (0, 0)),
        ],
        out_specs=[
            pl.BlockSpec((1, 1, BLK), lambda i: (i, 0, 0)),
            pl.BlockSpec((1, 1), lambda i: (0, 0)),
            pl.BlockSpec((K, D_in), lambda i: (0, 0)),
        ],
        out_shape=[
            jax.ShapeDtypeStruct((nblk, 1, BLK), jnp.int32),
            jax.ShapeDtypeStruct((1, 1), jnp.float32),
            jax.ShapeDtypeStruct((K, D_in), jnp.float32),
        ],
    )(xf, W_enc, b_enc.reshape(1, D_lat), codebook, W_dec, b_dec.reshape(1, D_in))

    q = q3.reshape(N)
    x_reco = _make_sc_gather(N, D_in, 1024)(table, q)
    loss = (loss_sum[0, 0] / jnp.float32(N * D_lat)).reshape(())
    return (x_reco.reshape(B, T, D_in), loss, q.reshape(B, T))


# speedup vs baseline: 1.7071x; 1.3480x over previous
import functools

import jax
import jax.numpy as jnp
from jax import lax
from jax.experimental import pallas as pl
from jax.experimental.pallas import tpu as pltpu
from jax.experimental.pallas import tpu_sc as plsc


def _tc_body(x_ref, we_ref, be_ref, cb_ref, wd_ref, bd_ref,
             q_ref, loss_ref, table_ref):
    i = pl.program_id(0)
    cb = cb_ref[...]
    K = cb.shape[0]

    @pl.when(i == 0)
    def _():
        table_ref[...] = jnp.dot(cb, wd_ref[...],
                                 preferred_element_type=jnp.float32) + bd_ref[...]

    x = x_ref[...]
    z = jnp.dot(x, we_ref[...], preferred_element_type=jnp.float32) + be_ref[...]
    zt = z.T
    csq = jnp.sum(cb * cb, axis=1)
    st = lax.dot_general(2.0 * cb, zt, (((1,), (0,)), ((), ())),
                         preferred_element_type=jnp.float32) - csq[:, None]
    G = K // 8
    s3 = st.reshape(G, 8, st.shape[1])
    best = s3[0]
    bj = jnp.zeros(best.shape, jnp.int32)
    for j in range(1, G):
        cur = s3[j]
        pred = cur > best
        best = jnp.where(pred, cur, best)
        bj = jnp.where(pred, jnp.int32(j), bj)
    sub = lax.broadcasted_iota(jnp.int32, best.shape, 0)
    cidx = bj * 8 + sub
    m = jnp.max(best, axis=0)
    q = jnp.min(jnp.where(best == m[None, :], cidx, jnp.int32(K)), axis=0)
    q_ref[0, 0, :] = q
    part = (jnp.sum(z * z) - jnp.sum(m)).reshape(1, 1)
    prev = jnp.where(i == 0, jnp.zeros((1, 1), jnp.float32), loss_ref[...])
    loss_ref[...] = prev + part


def _make_sc_gather(n_tokens, d_in, chunk):
    info = plsc.get_sparse_core_info()
    nc, ns = info.num_cores, info.num_subcores
    nw = nc * ns
    per_w = n_tokens // nw
    assert per_w % chunk == 0

    @functools.partial(
        pl.kernel,
        mesh=plsc.VectorSubcoreMesh(core_axis_name="c", subcore_axis_name="s"),
        out_type=jax.ShapeDtypeStruct((n_tokens, d_in), jnp.float32),
        scratch_types=[
            pltpu.VMEM((chunk,), jnp.int32),
            pltpu.VMEM((chunk, d_in), jnp.float32),
            pltpu.SemaphoreType.DMA,
        ],
        compiler_params=pltpu.CompilerParams(use_tc_tiling_on_sc=False),
    )
    def sc_gather(table_hbm, idx_hbm, out_hbm, idx_v, rows_v, sem):
        wid = lax.axis_index("s") * nc + lax.axis_index("c")
        base = wid * per_w
        for c in range(per_w // chunk):
            off = base + c * chunk
            pltpu.sync_copy(idx_hbm.at[pl.ds(off, chunk)], idx_v)
            pltpu.async_copy(table_hbm.at[idx_v], rows_v, sem).wait()
            pltpu.sync_copy(rows_v, out_hbm.at[pl.ds(off, chunk)])

    return sc_gather


def kernel(x, W_enc, b_enc, codebook, W_dec, b_dec):
    B, T, D_in = x.shape
    K, D_lat = codebook.shape
    N = B * T
    BLK = 2048
    nblk = N // BLK

    xf = x.reshape(N, D_in)
    q3, loss_sum, table = pl.pallas_call(
        _tc_body,
        grid=(nblk,),
        in_specs=[
            pl.BlockSpec((BLK, D_in), lambda i: (i, 0)),
            pl.BlockSpec((D_in, D_lat), lambda i: (0, 0)),
            pl.BlockSpec((1, D_lat), lambda i: (0, 0)),
            pl.BlockSpec((K, D_lat), lambda i: (0, 0)),
            pl.BlockSpec((D_lat, D_in), lambda i: (0, 0)),
            pl.BlockSpec((1, D_in), lambda i: (0, 0)),
        ],
        out_specs=[
            pl.BlockSpec((1, 1, BLK), lambda i: (i, 0, 0)),
            pl.BlockSpec((1, 1), lambda i: (0, 0)),
            pl.BlockSpec((K, D_in), lambda i: (0, 0)),
        ],
        out_shape=[
            jax.ShapeDtypeStruct((nblk, 1, BLK), jnp.int32),
            jax.ShapeDtypeStruct((1, 1), jnp.float32),
            jax.ShapeDtypeStruct((K, D_in), jnp.float32),
        ],
    )(xf, W_enc, b_enc.reshape(1, D_lat), codebook, W_dec, b_dec.reshape(1, D_in))

    q = q3.reshape(N)
    x_reco = _make_sc_gather(N, D_in, 1024)(table, q)
    loss = (loss_sum[0, 0] / jnp.float32(N * D_lat)).reshape(())
    return (x_reco.reshape(B, T, D_in), loss, q.reshape(B, T))
